# baseline (speedup 1.0000x reference)
import functools

import jax
import jax.numpy as jnp
from jax import lax
from jax.experimental import pallas as pl
from jax.experimental.pallas import tpu as pltpu
from jax.experimental.pallas import tpu_sc as plsc

B = 4
N = 10000
D = 128
E = 320000

NC = 2
NT = 16
EPT = E // NT
CH = 64
BCH = 16
NBLK = 20
EPT_PAD = NBLK * BCH * CH
NCHUNK = EPT_PAD // CH
RPT = 640
ZR = 80
NV = D // 16

_mesh = plsc.VectorSubcoreMesh(
    core_axis_name="c", subcore_axis_name="s", num_cores=NC, num_subcores=NT
)


@functools.partial(
    pl.kernel,
    out_type=jax.ShapeDtypeStruct((B, N, D), jnp.float32),
    mesh=_mesh,
    scratch_types=[
        pltpu.VMEM((2 * BCH, CH), jnp.int32),
        pltpu.VMEM((2 * BCH, CH), jnp.int32),
        pltpu.VMEM((2 * BCH, CH), jnp.float32),
        pltpu.VMEM((4 * CH, D), jnp.float32),
        pltpu.VMEM_SHARED((N, D), jnp.float32),
        pltpu.SemaphoreType.DMA,
        pltpu.SemaphoreType.DMA,
        pltpu.SemaphoreType.DMA,
    ],
)
def _aggregate(emb_hbm, col_hbm, row_hbm, val_hbm, out_hbm,
               colb, rowb, valb, rows, acc, sem_g, sem_i, sem_s):
    c = lax.axis_index("c")
    s = lax.axis_index("s")
    zvec = jnp.zeros((16,), jnp.float32)

    def zero_row(e, carry):
        for q in range(NV):
            rows[e, pl.ds(q * 16, 16)] = zvec
        return carry

    def blk_fetch(b, u, sync=False):
        sl = pl.ds((u & 1) * BCH, BCH)
        copy = pltpu.sync_copy if sync else (
            lambda src, dst: pltpu.async_copy(src, dst, sem_i))
        copy(col_hbm.at[b, s, u], colb.at[sl])
        copy(row_hbm.at[b, s, u], rowb.at[sl])
        copy(val_hbm.at[b, s, u], valb.at[sl])

    def blk_wait(b, u):
        sl = pl.ds((u & 1) * BCH, BCH)
        pltpu.make_async_copy(col_hbm.at[b, s, u], colb.at[sl], sem_i).wait()
        pltpu.make_async_copy(row_hbm.at[b, s, u], rowb.at[sl], sem_i).wait()
        pltpu.make_async_copy(val_hbm.at[b, s, u], valb.at[sl], sem_i).wait()

    def buf(j):
        return rows.at[pl.ds((j & 3) * CH, CH)]

    def gather_start(j, u, m):
        sidx = (u & 1) * BCH + m
        pltpu.async_copy(emb_hbm.at[colb.at[sidx]], buf(j), sem_g)

    def gather_wait(j, u, m):
        sidx = (u & 1) * BCH + m
        pltpu.make_async_copy(emb_hbm.at[colb.at[sidx]], buf(j), sem_g).wait()

    def scatter_start(j, u, m):
        sidx = (u & 1) * BCH + m
        pltpu.async_copy(buf(j), acc.at[rowb.at[sidx]], sem_s, add=True)

    def scatter_wait(j):
        pltpu.make_async_copy(buf(j), acc.at[rowb.at[0]], sem_s).wait()

    def scale(j, u, m):
        sidx = (u & 1) * BCH + m
        base = (j & 3) * CH

        def grp(g, gcarry):
            v16 = valb[sidx, pl.ds(g * 16, 16)]
            e0 = base + g * 16
            for k in range(16):
                e = e0 + k
                vecs = [rows[e, pl.ds(q * 16, 16)] for q in range(NV)]
                v = v16[k]
                for q in range(NV):
                    rows[e, pl.ds(q * 16, 16)] = vecs[q] * v
            return gcarry

        lax.fori_loop(0, CH // 16, grp, 0)

    def per_graph(i, carry):
        b = c * (B // NC) + i

        lax.fori_loop(0, ZR, zero_row, 0)
        zsrc = rows.at[pl.ds(0, ZR)]

        @pl.when(s < NT - 1)
        def _zero_full():
            for k in range(RPT // ZR):
                pltpu.sync_copy(zsrc, acc.at[pl.ds(s * RPT + k * ZR, ZR)])

        @pl.when(s == NT - 1)
        def _zero_tail():
            for k in range((N - (NT - 1) * RPT) // ZR):
                pltpu.sync_copy(
                    zsrc, acc.at[pl.ds((NT - 1) * RPT + k * ZR, ZR)])

        plsc.subcore_barrier()

        blk_fetch(b, 0, sync=True)
        gather_start(0, 0, 0)
        gather_start(1, 0, 1)

        def chunk(j, carry_um):
            u, m = carry_um
            gather_wait(j, u, m)

            @pl.when(j >= 2)
            def _free_next_buf():
                scatter_wait(j - 2)

            @pl.when(jnp.logical_and(m == 2, u + 1 < NBLK))
            def _prefetch_block():
                blk_fetch(b, u + 1)

            nj = j + 2
            nm = m + 2

            @pl.when(nm < BCH)
            def _next_gather_same_block():
                gather_start(nj, u, nm)

            @pl.when(jnp.logical_and(nm >= BCH, u + 1 < NBLK))
            def _next_gather_new_block():
                @pl.when(nm == BCH)
                def _wait_blk():
                    blk_wait(b, u + 1)

                gather_start(nj, u + 1, nm - BCH)

            scale(j, u, m)
            scatter_start(j, u, m)

            nu = jnp.where(m + 1 == BCH, u + 1, u)
            return (nu, jnp.where(m + 1 == BCH, 0, m + 1))

        lax.fori_loop(0, NCHUNK, chunk,
                      (jnp.int32(0), jnp.int32(0)))

        scatter_wait(NCHUNK - 2)
        scatter_wait(NCHUNK - 1)

        plsc.subcore_barrier()

        @pl.when(s < NT - 1)
        def _write_full():
            sl = pl.ds(s * RPT, RPT)
            pltpu.sync_copy(acc.at[sl], out_hbm.at[b, sl])

        @pl.when(s == NT - 1)
        def _write_tail():
            sl = pl.ds((NT - 1) * RPT, N - (NT - 1) * RPT)
            pltpu.sync_copy(acc.at[sl], out_hbm.at[b, sl])

        plsc.subcore_barrier()
        return carry

    lax.fori_loop(0, B // NC, per_graph, 0)


def kernel(last_embs, edge_index, edge_values):
    ei = edge_index.astype(jnp.int32)
    col = ei[:, 1, :] + (jnp.arange(B, dtype=jnp.int32) * N)[:, None]
    row = ei[:, 0, :]
    emb2 = last_embs.reshape(B * N, D)
    pad = ((0, 0), (0, 0), (0, EPT_PAD - EPT))
    col5 = jnp.pad(col.reshape(B, NT, EPT), pad).reshape(B, NT, NBLK, BCH, CH)
    row5 = jnp.pad(row.reshape(B, NT, EPT), pad).reshape(B, NT, NBLK, BCH, CH)
    val5 = jnp.pad(edge_values.reshape(B, NT, EPT), pad).reshape(
        B, NT, NBLK, BCH, CH)
    return _aggregate(emb2, col5, row5, val5)


# speedup vs baseline: 3.0295x; 3.0295x over previous
import functools

import jax
import jax.numpy as jnp
from jax import lax
from jax.experimental import pallas as pl
from jax.experimental.pallas import tpu as pltpu
from jax.experimental.pallas import tpu_sc as plsc

B = 4
N = 10000
D = 128
E = 320000

NC = 2
NT = 16
EPT = E // NT
CH = 80
NCHUNK = EPT // CH
RPT = 640
N_PAD = NT * RPT
NV = D // 16

_mesh = plsc.VectorSubcoreMesh(
    core_axis_name="c", subcore_axis_name="s", num_cores=NC, num_subcores=NT
)


@functools.partial(
    pl.kernel,
    out_type=jax.ShapeDtypeStruct((B, N, D), jnp.float32),
    mesh=_mesh,
    scratch_types=[
        pltpu.VMEM((16, CH), jnp.int32),
        pltpu.VMEM((8, CH), jnp.float32),
        pltpu.VMEM((4 * CH, D), jnp.float32),
        pltpu.VMEM_SHARED((N_PAD, D), jnp.float32),
        pltpu.SemaphoreType.DMA,
        pltpu.SemaphoreType.DMA,
        pltpu.SemaphoreType.DMA,
    ],
)
def _aggregate(emb_hbm, idx_hbm, val_hbm, out_hbm,
               idxr, valv, rows, acc, sem_g, sem_i, sem_s):
    c = lax.axis_index("c")
    s = lax.axis_index("s")
    zvec = jnp.zeros((16,), jnp.float32)

    def zero_row(e, carry):
        for q in range(NV):
            rows[e, pl.ds(q * 16, 16)] = zvec
        return carry

    def idx_fetch(b, j, sync=False):
        dst = idxr.at[pl.ds((j & 7) * 2, 2)]
        vdst = valv.at[j & 7]
        if sync:
            pltpu.sync_copy(idx_hbm.at[b, s, j], dst)
            pltpu.sync_copy(val_hbm.at[b, s, j], vdst)
        else:
            pltpu.async_copy(idx_hbm.at[b, s, j], dst, sem_i)
            pltpu.async_copy(val_hbm.at[b, s, j], vdst, sem_i)

    def idx_wait(b, j):
        dst = idxr.at[pl.ds((j & 7) * 2, 2)]
        vdst = valv.at[j & 7]
        pltpu.make_async_copy(idx_hbm.at[b, s, j], dst, sem_i).wait()
        pltpu.make_async_copy(val_hbm.at[b, s, j], vdst, sem_i).wait()

    def buf(j):
        return rows.at[pl.ds((j & 3) * CH, CH)]

    def gather_start(j):
        pltpu.async_copy(emb_hbm.at[idxr.at[(j & 7) * 2]], buf(j), sem_g)

    def gather_wait(j):
        pltpu.make_async_copy(emb_hbm.at[idxr.at[(j & 7) * 2]], buf(j), sem_g).wait()

    def scatter_start(j):
        pltpu.async_copy(buf(j), acc.at[idxr.at[(j & 7) * 2 + 1]], sem_s, add=True)

    def scatter_wait(j):
        pltpu.make_async_copy(buf(j), acc.at[idxr.at[0]], sem_s).wait()

    def scale(j):
        slot = j & 7
        base = (j & 3) * CH

        def grp(g, gcarry):
            v16 = valv[slot, pl.ds(g * 16, 16)]
            e0 = base + g * 16
            for k in range(16):
                e = e0 + k
                vecs = [rows[e, pl.ds(q * 16, 16)] for q in range(NV)]
                v = v16[k]
                for q in range(NV):
                    rows[e, pl.ds(q * 16, 16)] = vecs[q] * v
            return gcarry

        lax.fori_loop(0, CH // 16, grp, 0)

    def per_graph(i, carry):
        b = c * (B // NC) + i

        lax.fori_loop(0, CH, zero_row, 0)
        for k in range(RPT // CH):
            pltpu.sync_copy(rows.at[pl.ds(0, CH)],
                            acc.at[pl.ds(s * RPT + k * CH, CH)])

        plsc.subcore_barrier()

        idx_fetch(b, 0, sync=True)
        idx_fetch(b, 1, sync=True)
        gather_start(0)
        gather_start(1)
        idx_fetch(b, 2)

        def chunk(j, ccarry):
            gather_wait(j)

            @pl.when(j >= 2)
            def _free_next_buf():
                scatter_wait(j - 2)

            @pl.when(j + 2 < NCHUNK)
            def _next_gather():
                idx_wait(b, j + 2)
                gather_start(j + 2)

            @pl.when(j + 3 < NCHUNK)
            def _prefetch_idx():
                idx_fetch(b, j + 3)

            scale(j)
            scatter_start(j)
            return ccarry

        lax.fori_loop(0, NCHUNK, chunk, 0)

        scatter_wait(NCHUNK - 2)
        scatter_wait(NCHUNK - 1)

        plsc.subcore_barrier()

        @pl.when(s < NT - 1)
        def _write_full():
            sl = pl.ds(s * RPT, RPT)
            pltpu.sync_copy(acc.at[sl], out_hbm.at[b, sl])

        @pl.when(s == NT - 1)
        def _write_tail():
            sl = pl.ds((NT - 1) * RPT, N - (NT - 1) * RPT)
            pltpu.sync_copy(acc.at[sl], out_hbm.at[b, sl])

        plsc.subcore_barrier()
        return carry

    lax.fori_loop(0, B // NC, per_graph, 0)


def kernel(last_embs, edge_index, edge_values):
    ei = edge_index.astype(jnp.int32)
    col = ei[:, 1, :] + (jnp.arange(B, dtype=jnp.int32) * N)[:, None]
    row = ei[:, 0, :]
    emb2 = last_embs.reshape(B * N, D)
    col4 = col.reshape(B, NT, NCHUNK, CH)
    row4 = row.reshape(B, NT, NCHUNK, CH)
    val4 = edge_values.reshape(B, NT, NCHUNK, CH)
    idx2 = jnp.stack([col4, row4], axis=3)
    return _aggregate(emb2, idx2, val4)


# speedup vs baseline: 3.3183x; 1.0953x over previous
import functools

import jax
import jax.numpy as jnp
from jax import lax
from jax.experimental import pallas as pl
from jax.experimental.pallas import tpu as pltpu
from jax.experimental.pallas import tpu_sc as plsc

B = 4
N = 10000
D = 128
E = 320000

NC = 2
NT = 16
EPT = E // NT
CH = 80
NCHUNK = EPT // CH
RPT = 640
N_PAD = NT * RPT
NV = D // 16

_mesh = plsc.VectorSubcoreMesh(
    core_axis_name="c", subcore_axis_name="s", num_cores=NC, num_subcores=NT
)


@functools.partial(
    pl.kernel,
    out_type=jax.ShapeDtypeStruct((B, N, D), jnp.float32),
    mesh=_mesh,
    scratch_types=[
        pltpu.VMEM((8, CH), jnp.int32),
        pltpu.VMEM((8, CH), jnp.int32),
        pltpu.VMEM((8, CH), jnp.float32),
        pltpu.VMEM((4 * CH, D), jnp.float32),
        pltpu.VMEM_SHARED((N_PAD, D), jnp.float32),
        pltpu.SemaphoreType.DMA,
        pltpu.SemaphoreType.DMA,
        pltpu.SemaphoreType.DMA,
    ],
)
def _aggregate(emb_hbm, col_hbm, row_hbm, val_hbm, out_hbm,
               colv, rowv, valv, rows, acc, sem_g, sem_i, sem_s):
    c = lax.axis_index("c")
    s = lax.axis_index("s")
    zvec = jnp.zeros((16,), jnp.float32)

    def zero_row(e, carry):
        for q in range(NV):
            rows[e, pl.ds(q * 16, 16)] = zvec
        return carry

    def idx_fetch(b, j, sync=False):
        slot = j & 7
        copy = pltpu.sync_copy if sync else (
            lambda src, dst: pltpu.async_copy(src, dst, sem_i))
        copy(col_hbm.at[b, s, j], colv.at[slot])
        copy(row_hbm.at[b, s, j], rowv.at[slot])
        copy(val_hbm.at[b, s, j], valv.at[slot])

    def idx_wait(b, j):
        pltpu.make_async_copy(
            col_hbm.at[b, s, pl.ds(0, 3)], colv.at[pl.ds(0, 3)],
            sem_i).wait()

    def buf(j):
        return rows.at[pl.ds((j & 3) * CH, CH)]

    def gather_start(j):
        pltpu.async_copy(emb_hbm.at[colv.at[j & 7]], buf(j), sem_g)

    def gather_wait(j):
        pltpu.make_async_copy(emb_hbm.at[colv.at[j & 7]], buf(j), sem_g).wait()

    def scatter_start(j):
        pltpu.async_copy(buf(j), acc.at[rowv.at[j & 7]], sem_s, add=True)

    def scatter_wait(j):
        pltpu.make_async_copy(buf(j), acc.at[rowv.at[j & 7]], sem_s).wait()

    def scale(j):
        slot = j & 7
        base = (j & 3) * CH

        def grp(g, gcarry):
            v16 = valv[slot, pl.ds(g * 16, 16)]
            e0 = base + g * 16
            for k in range(16):
                e = e0 + k
                vecs = [rows[e, pl.ds(q * 16, 16)] for q in range(NV)]
                v = v16[k]
                for q in range(NV):
                    rows[e, pl.ds(q * 16, 16)] = vecs[q] * v
            return gcarry

        lax.fori_loop(0, CH // 16, grp, 0)

    def per_graph(i, carry):
        b = c * (B // NC) + i

        lax.fori_loop(0, CH, zero_row, 0)
        for k in range(RPT // CH):
            pltpu.sync_copy(rows.at[pl.ds(0, CH)],
                            acc.at[pl.ds(s * RPT + k * CH, CH)])

        plsc.subcore_barrier()

        idx_fetch(b, 0, sync=True)
        idx_fetch(b, 1, sync=True)
        gather_start(0)
        gather_start(1)
        idx_fetch(b, 2)

        def chunk(j, ccarry):
            gather_wait(j)

            @pl.when(j >= 2)
            def _free_next_buf():
                scatter_wait(j - 2)

            @pl.when(j + 2 < NCHUNK)
            def _next_gather():
                idx_wait(b, j + 2)
                gather_start(j + 2)

            @pl.when(j + 3 < NCHUNK)
            def _prefetch_idx():
                idx_fetch(b, j + 3)

            scale(j)
            scatter_start(j)
            return ccarry

        lax.fori_loop(0, NCHUNK, chunk, 0)

        scatter_wait(NCHUNK - 2)
        scatter_wait(NCHUNK - 1)

        plsc.subcore_barrier()

        @pl.when(s < NT - 1)
        def _write_full():
            sl = pl.ds(s * RPT, RPT)
            pltpu.sync_copy(acc.at[sl], out_hbm.at[b, sl])

        @pl.when(s == NT - 1)
        def _write_tail():
            sl = pl.ds((NT - 1) * RPT, N - (NT - 1) * RPT)
            pltpu.sync_copy(acc.at[sl], out_hbm.at[b, sl])

        plsc.subcore_barrier()
        return carry

    lax.fori_loop(0, B // NC, per_graph, 0)


def kernel(last_embs, edge_index, edge_values):
    ei = edge_index.astype(jnp.int32)
    col = ei[:, 1, :] + (jnp.arange(B, dtype=jnp.int32) * N)[:, None]
    row = ei[:, 0, :]
    emb2 = last_embs.reshape(B * N, D)
    col4 = col.reshape(B, NT, NCHUNK, CH)
    row4 = row.reshape(B, NT, NCHUNK, CH)
    val4 = edge_values.reshape(B, NT, NCHUNK, CH)
    return _aggregate(emb2, col4, row4, val4)
